# Initial kernel scaffold; baseline (speedup 1.0000x reference)
import jax
import jax.numpy as jnp
from jax.experimental import pallas as pl


def kernel(x_author, x_paper, W_l_ap, b_l_ap, W_r_ap, W_l_pa, b_l_pa, W_r_pa, W_gat, att_src, att_dst, b_gat, prelu_a, edge_index_ap, edge_index_pa, perm):
    raise NotImplementedError("write your pallas kernel here")


# speedup vs baseline: 9.2272x; 9.2272x over previous
import functools

import jax
import jax.numpy as jnp
from jax import lax
from jax.experimental import pallas as pl
from jax.experimental.pallas import tpu as pltpu
from jax.experimental.pallas import tpu_sc as plsc

NA = 10000
NP = 10000
NE = 320000
D = 128

NC = 2
NS = 16
NW = NC * NS
L = 16

NROW = 10240
JROW = 10000
TE = NROW
EPAD = NW * TE
CHUNK = 128
NCHUNK = TE // CHUNK
RPT = NROW // NS
PCH = 64
PPT = NROW // NW

_mesh = plsc.VectorSubcoreMesh(
    core_axis_name="c", subcore_axis_name="s", num_cores=NC, num_subcores=NS)

_f32 = jnp.float32
_i32 = jnp.int32


def _sage_body(src_hbm, dst_hbm, xp_hbm, out_sum, out_cnt,
               sidx, didx, rows, cnt, acc):
  cid = lax.axis_index("c")
  sid = lax.axis_index("s")
  wid = sid * NC + cid
  zero16 = jnp.zeros((L,), _f32)

  @pl.loop(0, NROW // L)
  def _(i):
    cnt[pl.ds(i * L, L)] = zero16

  @pl.loop(0, CHUNK)
  def _(r):
    for c in range(D // L):
      rows[r, pl.ds(c * L, L)] = zero16

  for k in range(RPT // CHUNK):
    pltpu.sync_copy(rows, acc.at[pl.ds(sid * RPT + k * CHUNK, CHUNK)])
  plsc.subcore_barrier()

  ones16 = jnp.full((L,), 1.0, _f32)
  base = wid * TE

  @pl.loop(0, NCHUNK)
  def _(i):
    off = base + i * CHUNK
    pltpu.sync_copy(src_hbm.at[pl.ds(off, CHUNK)], sidx)
    pltpu.sync_copy(dst_hbm.at[pl.ds(off, CHUNK)], didx)
    pltpu.sync_copy(xp_hbm.at[sidx], rows)
    for j in range(CHUNK // L):
      d16 = didx[pl.ds(j * L, L)]
      plsc.addupdate_scatter(cnt, [d16], ones16)
    pltpu.sync_copy(rows, acc.at[didx], add=True)

  plsc.subcore_barrier()
  pltpu.sync_copy(cnt, out_cnt.at[wid])
  for k in range(RPT // CHUNK):
    sl = pl.ds(sid * RPT + k * CHUNK, CHUNK)
    pltpu.sync_copy(acc.at[sl], out_sum.at[cid, sl])


def _sage_agg(src_pa, dst_pa, x_paper):
  return pl.kernel(
      _sage_body,
      out_type=[jax.ShapeDtypeStruct((NC, NROW, D), _f32),
                jax.ShapeDtypeStruct((NW, NROW), _f32)],
      mesh=_mesh,
      compiler_params=pltpu.CompilerParams(needs_layout_passes=False),
      scratch_types=[
          pltpu.VMEM((CHUNK,), _i32),
          pltpu.VMEM((CHUNK,), _i32),
          pltpu.VMEM((CHUNK, D), _f32),
          pltpu.VMEM((NROW,), _f32),
          pltpu.VMEM_SHARED((NROW, D), _f32),
      ],
  )(src_pa, dst_pa, x_paper)


def _gatw_body(src_hbm, pap_hbm, p2a_hbm, as_hbm, ad_hbm, perm_hbm,
               out_wp, out_wn, out_gn, out_dd, out_denp, out_denn,
               sbuf, pbuf, wpb, wnb, gnb, ddb, p2a_v, as_v, ad_v, perm_v,
               denp_v, denn_v):
  cid = lax.axis_index("c")
  sid = lax.axis_index("s")
  wid = sid * NC + cid
  zero16 = jnp.zeros((L,), _f32)

  pltpu.sync_copy(p2a_hbm, p2a_v)
  pltpu.sync_copy(as_hbm, as_v)
  pltpu.sync_copy(ad_hbm, ad_v)
  pltpu.sync_copy(perm_hbm, perm_v)

  @pl.loop(0, NROW // L)
  def _(i):
    denp_v[pl.ds(i * L, L)] = zero16
    denn_v[pl.ds(i * L, L)] = zero16

  base = wid * TE

  @pl.loop(0, NCHUNK)
  def _(i):
    off = base + i * CHUNK
    pltpu.sync_copy(src_hbm.at[pl.ds(off, CHUNK)], sbuf)
    pltpu.sync_copy(pap_hbm.at[pl.ds(off, CHUNK)], pbuf)
    for j in range(CHUNK // L):
      sl = pl.ds(j * L, L)
      s16 = sbuf[sl]
      p16 = pbuf[sl]
      mapped = plsc.load_gather(p2a_v, [p16])
      mask = mapped >= 0
      d0 = jnp.where(mask, mapped, 0)
      dsc = jnp.where(mask, mapped, JROW)

      def edge_w(si, di):
        e = plsc.load_gather(as_v, [si]) + plsc.load_gather(ad_v, [di])
        e = jnp.where(e >= 0.0, e, e * 0.2)
        return jnp.where(mask, jnp.exp(e), 0.0)

      wp = edge_w(s16, d0)
      sgn = plsc.load_gather(perm_v, [s16])
      dgn = plsc.load_gather(perm_v, [d0])
      wn = edge_w(sgn, dgn)
      plsc.addupdate_scatter(denp_v, [dsc], wp, mask=mask)
      plsc.addupdate_scatter(denn_v, [dsc], wn, mask=mask)
      wpb[sl] = wp
      wnb[sl] = wn
      gnb[sl] = sgn
      ddb[sl] = dsc
    pltpu.sync_copy(wpb, out_wp.at[pl.ds(off, CHUNK)])
    pltpu.sync_copy(wnb, out_wn.at[pl.ds(off, CHUNK)])
    pltpu.sync_copy(gnb, out_gn.at[pl.ds(off, CHUNK)])
    pltpu.sync_copy(ddb, out_dd.at[pl.ds(off, CHUNK)])

  pltpu.sync_copy(denp_v, out_denp.at[wid])
  pltpu.sync_copy(denn_v, out_denn.at[wid])


def _gat_weights(src_ap, pap_ap, p2a_pad, a_s, a_d, perm_tab):
  return pl.kernel(
      _gatw_body,
      out_type=[jax.ShapeDtypeStruct((EPAD,), _f32),
                jax.ShapeDtypeStruct((EPAD,), _f32),
                jax.ShapeDtypeStruct((EPAD,), _i32),
                jax.ShapeDtypeStruct((EPAD,), _i32),
                jax.ShapeDtypeStruct((NW, NROW), _f32),
                jax.ShapeDtypeStruct((NW, NROW), _f32)],
      mesh=_mesh,
      compiler_params=pltpu.CompilerParams(needs_layout_passes=False),
      scratch_types=[
          pltpu.VMEM((CHUNK,), _i32),
          pltpu.VMEM((CHUNK,), _i32),
          pltpu.VMEM((CHUNK,), _f32),
          pltpu.VMEM((CHUNK,), _f32),
          pltpu.VMEM((CHUNK,), _i32),
          pltpu.VMEM((CHUNK,), _i32),
          pltpu.VMEM((NROW,), _i32),
          pltpu.VMEM((NA,), _f32),
          pltpu.VMEM((NA,), _f32),
          pltpu.VMEM((NA,), _i32),
          pltpu.VMEM((NROW,), _f32),
          pltpu.VMEM((NROW,), _f32),
      ],
  )(src_ap, pap_ap, p2a_pad, a_s, a_d, perm_tab)


def _gatr_body(gidx_hbm, didx_hbm, w_hbm, h_hbm, out_num,
               gbuf, dbuf, wbuf, rows, acc):
  cid = lax.axis_index("c")
  sid = lax.axis_index("s")
  wid = sid * NC + cid
  zero16 = jnp.zeros((L,), _f32)

  @pl.loop(0, CHUNK)
  def _(r):
    for c in range(D // L):
      rows[r, pl.ds(c * L, L)] = zero16

  for k in range(RPT // CHUNK):
    pltpu.sync_copy(rows, acc.at[pl.ds(sid * RPT + k * CHUNK, CHUNK)])
  plsc.subcore_barrier()

  base = wid * TE

  @pl.loop(0, NCHUNK)
  def _(i):
    off = base + i * CHUNK
    pltpu.sync_copy(gidx_hbm.at[pl.ds(off, CHUNK)], gbuf)
    pltpu.sync_copy(didx_hbm.at[pl.ds(off, CHUNK)], dbuf)
    pltpu.sync_copy(w_hbm.at[pl.ds(off, CHUNK)], wbuf)
    pltpu.sync_copy(h_hbm.at[gbuf], rows)

    @pl.loop(0, CHUNK // L)
    def _(g):
      w16 = wbuf[pl.ds(g * L, L)]
      for kk in range(L):
        wr = w16[kk]
        r = g * L + kk
        for c in range(D // L):
          sl2 = pl.ds(c * L, L)
          rows[r, sl2] = rows[r, sl2] * wr

    pltpu.sync_copy(rows, acc.at[dbuf], add=True)

  plsc.subcore_barrier()
  for k in range(RPT // CHUNK):
    sl = pl.ds(sid * RPT + k * CHUNK, CHUNK)
    pltpu.sync_copy(acc.at[sl], out_num.at[cid, sl])


def _gat_rows(gidx, didx, w, h):
  return pl.kernel(
      _gatr_body,
      out_type=[jax.ShapeDtypeStruct((NC, NROW, D), _f32)],
      mesh=_mesh,
      compiler_params=pltpu.CompilerParams(needs_layout_passes=False),
      scratch_types=[
          pltpu.VMEM((CHUNK,), _i32),
          pltpu.VMEM((CHUNK,), _i32),
          pltpu.VMEM((CHUNK,), _f32),
          pltpu.VMEM((CHUNK, D), _f32),
          pltpu.VMEM_SHARED((NROW, D), _f32),
      ],
  )(gidx, didx, w, h)


def _perm_body(h_hbm, ws_hbm, perm_hbm, out_hp, out_wsp,
               ibuf, rows, wtab, wout):
  cid = lax.axis_index("c")
  sid = lax.axis_index("s")
  wid = sid * NC + cid
  pltpu.sync_copy(ws_hbm, wtab)
  base = wid * PPT

  @pl.loop(0, PPT // PCH)
  def _(k):
    off = base + k * PCH
    pltpu.sync_copy(perm_hbm.at[pl.ds(off, PCH)], ibuf)
    pltpu.sync_copy(h_hbm.at[ibuf], rows)
    pltpu.sync_copy(rows, out_hp.at[pl.ds(off, PCH)])
    for g in range(PCH // L):
      sl = pl.ds(g * L, L)
      wout[sl] = plsc.load_gather(wtab, [ibuf[sl]])
    pltpu.sync_copy(wout, out_wsp.at[pl.ds(off, PCH)])


def _permute(h, ws_flat, perm_pad):
  return pl.kernel(
      _perm_body,
      out_type=[jax.ShapeDtypeStruct((NROW, D), _f32),
                jax.ShapeDtypeStruct((NROW,), _f32)],
      mesh=_mesh,
      compiler_params=pltpu.CompilerParams(needs_layout_passes=False),
      scratch_types=[
          pltpu.VMEM((PCH,), _i32),
          pltpu.VMEM((PCH, D), _f32),
          pltpu.VMEM((NA,), _f32),
          pltpu.VMEM((PCH,), _f32),
      ],
  )(h, ws_flat, perm_pad)


_BLK = 2048
_GRID = NROW // _BLK


def _tc1_body(sum_ref, cnt_ref, xa_ref, wl_ref, bl_ref, wr_ref, wg_ref,
              as_ref, ad_ref, h_ref, asv_ref, adv_ref, ws_ref):
  dn = (((1,), (1,)), ((), ()))
  s = sum_ref[0] + sum_ref[1]
  c = jnp.sum(cnt_ref[...], axis=0)
  mean = s / jnp.maximum(c, 1.0)[:, None]
  ha = lax.dot_general(mean, wl_ref[...], dn, preferred_element_type=_f32)
  ha = ha + bl_ref[...]
  ha = ha + lax.dot_general(xa_ref[...], wr_ref[...], dn,
                            preferred_element_type=_f32)
  ha = jnp.where(ha >= 0.0, ha, 0.01 * ha)
  h = lax.dot_general(ha, wg_ref[...], dn, preferred_element_type=_f32)
  h_ref[...] = h
  dv = (((1,), (0,)), ((), ()))
  a_s = lax.dot_general(h, as_ref[...], dv, preferred_element_type=_f32)
  a_d = lax.dot_general(h, ad_ref[...], dv, preferred_element_type=_f32)
  asv_ref[...] = a_s
  adv_ref[...] = a_d
  e = a_s + a_d
  e = jnp.where(e >= 0.0, e, 0.2 * e)
  ws_ref[...] = jnp.exp(e)


def _tc1(sum2, cnt32, xa_pad, Wl, bl, Wr, Wg, att_s, att_d):
  col = pl.BlockSpec((_BLK, 1), lambda i: (i, 0))
  full = pl.BlockSpec((D, D), lambda i: (0, 0))
  return pl.pallas_call(
      _tc1_body,
      grid=(_GRID,),
      in_specs=[pl.BlockSpec((NC, _BLK, D), lambda i: (0, i, 0)),
                pl.BlockSpec((NW, _BLK), lambda i: (0, i)),
                pl.BlockSpec((_BLK, D), lambda i: (i, 0)),
                full,
                pl.BlockSpec((1, D), lambda i: (0, 0)),
                full, full,
                pl.BlockSpec((D, 1), lambda i: (0, 0)),
                pl.BlockSpec((D, 1), lambda i: (0, 0))],
      out_specs=[pl.BlockSpec((_BLK, D), lambda i: (i, 0)), col, col, col],
      out_shape=[jax.ShapeDtypeStruct((NROW, D), _f32),
                 jax.ShapeDtypeStruct((NROW, 1), _f32),
                 jax.ShapeDtypeStruct((NROW, 1), _f32),
                 jax.ShapeDtypeStruct((NROW, 1), _f32)],
  )(sum2, cnt32, xa_pad, Wl, bl, Wr, Wg, att_s, att_d)


def _tc2_body(np_ref, dp_ref, nn_ref, dn_ref, h_ref, ws_ref, hp_ref, wsp_ref,
              bg_ref, pa_ref, pos_ref, neg_ref, sum_ref):
  i = pl.program_id(0)
  a = pa_ref[0, 0]

  def fin(nref, dref, hv, wv):
    num = nref[0] + nref[1] + wv * hv
    den = jnp.sum(dref[...], axis=0)[:, None] + wv
    o = num / den + bg_ref[...]
    return jnp.where(o >= 0.0, o, a * o)

  pos = fin(np_ref, dp_ref, h_ref[...], ws_ref[...])
  neg = fin(nn_ref, dn_ref, hp_ref[...], wsp_ref[...])
  pos_ref[...] = pos
  neg_ref[...] = neg
  rid = lax.broadcasted_iota(_i32, (_BLK, 1), 0) + i * _BLK
  part = jnp.sum(jnp.where(rid < NA, pos, 0.0), axis=0, keepdims=True)

  @pl.when(i == 0)
  def _():
    sum_ref[...] = jnp.zeros_like(sum_ref)

  sum_ref[...] += part

  @pl.when(i == _GRID - 1)
  def _():
    sum_ref[...] = sum_ref[...] * (1.0 / NA)


def _tc2(nump, denp, numn, denn, h, ws, hp, wsp, bg, pa):
  col = pl.BlockSpec((_BLK, 1), lambda i: (i, 0))
  mat = pl.BlockSpec((_BLK, D), lambda i: (i, 0))
  return pl.pallas_call(
      _tc2_body,
      grid=(_GRID,),
      in_specs=[pl.BlockSpec((NC, _BLK, D), lambda i: (0, i, 0)),
                pl.BlockSpec((NW, _BLK), lambda i: (0, i)),
                pl.BlockSpec((NC, _BLK, D), lambda i: (0, i, 0)),
                pl.BlockSpec((NW, _BLK), lambda i: (0, i)),
                mat, col, mat, col,
                pl.BlockSpec((1, D), lambda i: (0, 0)),
                pl.BlockSpec((1, 1), lambda i: (0, 0))],
      out_specs=[mat, mat, pl.BlockSpec((1, D), lambda i: (0, 0))],
      out_shape=[jax.ShapeDtypeStruct((NROW, D), _f32),
                 jax.ShapeDtypeStruct((NROW, D), _f32),
                 jax.ShapeDtypeStruct((1, D), _f32)],
  )(nump, denp, numn, denn, h, ws, hp, wsp, bg, pa)


@jax.jit
def kernel(x_author, x_paper, W_l_ap, b_l_ap, W_r_ap, W_l_pa, b_l_pa, W_r_pa,
           W_gat, att_src, att_dst, b_gat, prelu_a, edge_index_ap,
           edge_index_pa, perm):
  src_pa = edge_index_pa[0].astype(_i32)
  dst_pa = edge_index_pa[1].astype(_i32)
  src_ap = edge_index_ap[0].astype(_i32)
  pap_ap = edge_index_ap[1].astype(_i32)
  permc = perm.astype(_i32)

  npad = EPAD - NE
  src_pa_p = jnp.concatenate([src_pa, jnp.zeros((npad,), _i32)])
  dst_pa_p = jnp.concatenate([dst_pa, jnp.full((npad,), JROW, _i32)])
  src_ap_p = jnp.concatenate([src_ap, jnp.zeros((npad,), _i32)])
  pap_ap_p = jnp.concatenate([pap_ap, jnp.full((npad,), JROW, _i32)])

  p2a = jnp.full((NP,), -1, _i32).at[src_pa].set(dst_pa)
  p2a_pad = jnp.concatenate([p2a, jnp.full((NROW - NP,), -1, _i32)])
  perm_pad = jnp.concatenate([permc, jnp.zeros((NROW - NA,), _i32)])
  xa_pad = jnp.concatenate([x_author, jnp.zeros((NROW - NA, D), _f32)])

  sum2, cnt32 = _sage_agg(src_pa_p, dst_pa_p, x_paper)
  h, a_s2, a_d2, ws2 = _tc1(sum2, cnt32, xa_pad, W_l_pa,
                            b_l_pa.reshape(1, D), W_r_pa, W_gat,
                            att_src.reshape(D, 1), att_dst.reshape(D, 1))
  a_s = a_s2[:NA, 0]
  a_d = a_d2[:NA, 0]
  ws_flat = ws2[:NA, 0]
  hp, wsp = _permute(h, ws_flat, perm_pad)
  wp, wn, gn, dd, denp, denn = _gat_weights(src_ap_p, pap_ap_p, p2a_pad,
                                            a_s, a_d, permc)
  nump = _gat_rows(src_ap_p, dd, wp, h)[0]
  numn = _gat_rows(gn, dd, wn, h)[0]
  pos_f, neg_f, summ = _tc2(nump, denp, numn, denn, h, ws2, hp,
                            wsp.reshape(NROW, 1), b_gat.reshape(1, D),
                            prelu_a.reshape(1, 1))
  return pos_f[:NA], neg_f[:NA], summ.reshape(D)
